# Initial kernel scaffold; baseline (speedup 1.0000x reference)
import jax
import jax.numpy as jnp
from jax.experimental import pallas as pl


def kernel(x, s, edge_index, W_emb, b_emb, gin_W1, gin_b1, gin_W2, gin_b2, gcn_W, gcn_b, Whp_W, Whp_b):
    raise NotImplementedError("write your pallas kernel here")


# speedup vs baseline: 4.4111x; 4.4111x over previous
import functools

import jax
import jax.numpy as jnp
from jax import lax
from jax.experimental import pallas as pl
from jax.experimental.pallas import tpu as pltpu
from jax.experimental.pallas import tpu_sc as plsc

NC = 2
NS = 16
CH = 80


def _row_split(n):
    rpt = (n // NS) // 8 * 8
    last = n - (NS - 1) * rpt
    assert last % 8 == 0 and 0 < last
    return rpt, last


def _span(sid, copy, rpt, last):
    @pl.when(sid < NS - 1)
    def _():
        copy(rpt)

    @pl.when(sid == NS - 1)
    def _():
        copy(last)


def _sc_mesh():
    return plsc.VectorSubcoreMesh(core_axis_name="c", subcore_axis_name="s",
                                  num_cores=NC, num_subcores=NS)


def _make_deg_kernel(n, e, w):
    ept = e // (NC * NS)
    nch = ept // CH
    assert ept * NC * NS == e and nch * CH == ept
    rpt, last = _row_split(n)

    @functools.partial(
        pl.kernel,
        out_type=jax.ShapeDtypeStruct((NC, n, w), jnp.float32),
        mesh=_sc_mesh(),
        scratch_types=[
            pltpu.VMEM_SHARED((n, w), jnp.float32),
            pltpu.VMEM((CH, w), jnp.float32),
            pltpu.VMEM((1, CH), jnp.int32),
        ],
    )
    def deg_kernel(dst_hbm, zeros_hbm, ones_hbm, out_hbm, acc, ones_v, didx):
        cid = lax.axis_index("c")
        sid = lax.axis_index("s")
        row0 = pl.multiple_of(sid * rpt, 8)
        ebase = cid * (e // NC) + sid * ept
        pltpu.sync_copy(ones_hbm, ones_v)
        _span(sid, lambda nr: pltpu.sync_copy(
            zeros_hbm.at[pl.ds(row0, nr)], acc.at[pl.ds(row0, nr)]),
            rpt, last)
        plsc.subcore_barrier()

        def body(j, _):
            base = pl.multiple_of(ebase + j * CH, 8)
            pltpu.sync_copy(dst_hbm.at[pl.ds(base, CH)], didx.at[0])
            pltpu.sync_copy(ones_v, acc.at[didx.at[0]], add=True)
            return 0
        lax.fori_loop(0, nch, body, 0)
        plsc.subcore_barrier()
        _span(sid, lambda nr: pltpu.sync_copy(
            acc.at[pl.ds(row0, nr)], out_hbm.at[cid, pl.ds(row0, nr)]),
            rpt, last)

    return deg_kernel


def _make_seg3_kernel(n, e, h):
    ept = e // (NC * NS)
    nch = ept // CH
    assert ept * NC * NS == e and nch * CH == ept
    rpt, last = _row_split(n)

    @functools.partial(
        pl.kernel,
        out_type=[jax.ShapeDtypeStruct((NC, n, h), jnp.float32)] * 3,
        mesh=_sc_mesh(),
        scratch_types=[
            pltpu.VMEM_SHARED((n, h), jnp.float32),
            pltpu.VMEM((CH, h), jnp.float32),
            pltpu.VMEM((1, CH), jnp.int32),
            pltpu.VMEM((1, CH), jnp.int32),
            pltpu.SemaphoreType.DMA,
        ],
    )
    def seg3_kernel(src_hbm, dst_hbm, x_hbm, s_hbm, t_hbm, zeros_hbm,
                    ox_hbm, os_hbm, ot_hbm, acc, rows, sidx, didx, sem):
        cid = lax.axis_index("c")
        sid = lax.axis_index("s")
        row0 = pl.multiple_of(sid * rpt, 8)
        ebase = cid * (e // NC) + sid * ept

        def one_pass(table_hbm, out_hbm):
            _span(sid, lambda nr: pltpu.sync_copy(
                zeros_hbm.at[pl.ds(row0, nr)], acc.at[pl.ds(row0, nr)]),
                rpt, last)
            plsc.subcore_barrier()

            def body(j, _):
                base = pl.multiple_of(ebase + j * CH, 8)
                pltpu.sync_copy(src_hbm.at[pl.ds(base, CH)], sidx.at[0])
                pltpu.sync_copy(dst_hbm.at[pl.ds(base, CH)], didx.at[0])
                pltpu.async_copy(table_hbm.at[sidx.at[0]], rows, sem).wait()
                pltpu.sync_copy(rows, acc.at[didx.at[0]], add=True)
                return 0
            lax.fori_loop(0, nch, body, 0)
            plsc.subcore_barrier()
            _span(sid, lambda nr: pltpu.sync_copy(
                acc.at[pl.ds(row0, nr)], out_hbm.at[cid, pl.ds(row0, nr)]),
                rpt, last)
            plsc.subcore_barrier()

        one_pass(x_hbm, ox_hbm)
        one_pass(s_hbm, os_hbm)
        one_pass(t_hbm, ot_hbm)

    return seg3_kernel


_R = 1000


def _tc_init_kernel(sraw_ref, wemb_ref, bemb_ref, deg_ref,
                    s0_ref, t0_ref, dinv_ref):
    deg = deg_ref[0, :, 0:1] + deg_ref[1, :, 0:1] + 1.0
    dinv = lax.rsqrt(jnp.maximum(deg, 1e-12))
    dinvb = jnp.broadcast_to(dinv, (deg.shape[0], s0_ref.shape[-1]))
    s0 = jnp.dot(sraw_ref[...], wemb_ref[...],
                 preferred_element_type=jnp.float32) + bemb_ref[...]
    s0_ref[...] = s0
    t0_ref[...] = dinvb * s0
    dinv_ref[...] = dinvb


def _tc_layer_kernel(x_ref, s_ref, ax_ref, as_ref, at_ref, dinv_ref,
                     w1x_ref, w1y_ref, b1_ref, w2_ref, b2_ref,
                     wg_ref, bg_ref,
                     xo_ref, so_ref, to_ref):
    dinv = dinv_ref[...]
    gx = x_ref[...] + ax_ref[0] + ax_ref[1]
    gs = s_ref[...] + as_ref[0] + as_ref[1]
    h = jnp.dot(gx, w1x_ref[...], preferred_element_type=jnp.float32)
    h = h + jnp.dot(gs, w1y_ref[...], preferred_element_type=jnp.float32)
    h = jnp.maximum(h + b1_ref[...], 0.0)
    xo = jnp.dot(h, w2_ref[...], preferred_element_type=jnp.float32)
    xo_ref[...] = jnp.maximum(xo + b2_ref[...], 0.0)
    u = dinv * (at_ref[0] + at_ref[1]) + dinv * dinv * s_ref[...]
    so = jnp.tanh(jnp.dot(u, wg_ref[...],
                          preferred_element_type=jnp.float32) + bg_ref[...])
    so_ref[...] = so
    to_ref[...] = dinv * so


def _tc_final_kernel(x_ref, s_ref, wx_ref, wy_ref, b_ref, out_ref):
    o = jnp.dot(x_ref[...], wx_ref[...], preferred_element_type=jnp.float32)
    o = o + jnp.dot(s_ref[...], wy_ref[...], preferred_element_type=jnp.float32)
    out_ref[...] = o + b_ref[...]


def _rows(r, w):
    return pl.BlockSpec((r, w), lambda i: (i, 0))


def _parts(r, w):
    return pl.BlockSpec((NC, r, w), lambda i: (0, i, 0))


def _full(shape):
    return pl.BlockSpec(shape, lambda i: tuple(0 for _ in shape))


def kernel(x, s, edge_index, W_emb, b_emb, gin_W1, gin_b1, gin_W2, gin_b2,
           gcn_W, gcn_b, Whp_W, Whp_b):
    n, h = x.shape
    nse = s.shape[1]
    e = edge_index.shape[1]
    nl = gin_W1.shape[0]
    src = edge_index[0]
    dst = edge_index[1]
    zeros_rows = jnp.zeros((n, h), jnp.float32)
    ones_chunk = jnp.ones((CH, h), jnp.float32)

    deg_parts = _make_deg_kernel(n, e, h)(dst, zeros_rows, ones_chunk)
    seg3 = _make_seg3_kernel(n, e, h)

    grid = (n // _R,)
    s0, t0, dinvb = pl.pallas_call(
        _tc_init_kernel,
        grid=grid,
        in_specs=[_rows(_R, nse), _full((nse, h)), _full((1, h)),
                  _parts(_R, h)],
        out_specs=[_rows(_R, h), _rows(_R, h), _rows(_R, h)],
        out_shape=[jax.ShapeDtypeStruct((n, h), jnp.float32)] * 3,
    )(s, W_emb, b_emb.reshape(1, h), deg_parts)

    xs, ss, ts = x, s0, t0
    for l in range(nl):
        ax, as_, at = seg3(src, dst, xs, ss, ts, zeros_rows)
        xs, ss, ts = pl.pallas_call(
            _tc_layer_kernel,
            grid=grid,
            in_specs=[_rows(_R, h), _rows(_R, h),
                      _parts(_R, h), _parts(_R, h), _parts(_R, h),
                      _rows(_R, h),
                      _full((h, h)), _full((h, h)), _full((1, h)),
                      _full((h, h)), _full((1, h)),
                      _full((h, h)), _full((1, h))],
            out_specs=[_rows(_R, h)] * 3,
            out_shape=[jax.ShapeDtypeStruct((n, h), jnp.float32)] * 3,
        )(xs, ss, ax, as_, at, dinvb,
          gin_W1[l, :h], gin_W1[l, h:], gin_b1[l].reshape(1, h),
          gin_W2[l], gin_b2[l].reshape(1, h),
          gcn_W[l], gcn_b[l].reshape(1, h))

    out = pl.pallas_call(
        _tc_final_kernel,
        grid=grid,
        in_specs=[_rows(_R, h), _rows(_R, h),
                  _full((h, h)), _full((h, h)), _full((1, h))],
        out_specs=_rows(_R, h),
        out_shape=jax.ShapeDtypeStruct((n, h), jnp.float32),
    )(xs, ss, Whp_W[:h], Whp_W[h:], Whp_b.reshape(1, h))
    return out
